# baseline (speedup 1.0000x reference)
import jax
import jax.numpy as jnp
from jax.experimental import pallas as pl
from jax.experimental.pallas import tpu as pltpu

B, D, H, W = 4, 96, 384, 384
P = H * W
PR = P // 128
RB = 72
NB = PR // RB


def _body(s_ref, t_ref, m_ref, x_ref, sum_ref, cnt_ref):
    step = pl.program_id(0) * pl.num_programs(1) + pl.program_id(1)

    @pl.when(step == 0)
    def _init():
        sum_ref[...] = jnp.zeros_like(sum_ref)
        cnt_ref[...] = jnp.zeros_like(cnt_ref)

    s = s_ref[0]
    t = t_ref[0]
    dot = jnp.sum(s * t, axis=0)
    ns2 = jnp.sum(s * s, axis=0)
    nt2 = jnp.sum(t * t, axis=0)
    denom = jnp.maximum(jnp.sqrt(ns2) * jnp.sqrt(nt2), 1e-8)
    loss_map = 1.0 - dot / denom

    valid = (x_ref[0] != 0.0) & (m_ref[0] == 0)
    vf = valid.astype(jnp.float32)
    sum_ref[...] += jnp.sum(loss_map * vf, keepdims=True).reshape(1, 1)
    cnt_ref[...] += jnp.sum(vf, keepdims=True).reshape(1, 1)


def kernel(student_feats, teacher_feats, mask, original_x):
    s = student_feats.reshape(B, D, PR, 128)
    t = teacher_feats.reshape(B, D, PR, 128)
    m = mask.reshape(B, PR, 128).astype(jnp.float32)
    x = original_x.reshape(B, PR, 128)

    sums, cnts = pl.pallas_call(
        _body,
        grid=(B, NB),
        in_specs=[
            pl.BlockSpec((1, D, RB, 128), lambda b, j: (b, 0, j, 0)),
            pl.BlockSpec((1, D, RB, 128), lambda b, j: (b, 0, j, 0)),
            pl.BlockSpec((1, RB, 128), lambda b, j: (b, j, 0)),
            pl.BlockSpec((1, RB, 128), lambda b, j: (b, j, 0)),
        ],
        out_specs=[
            pl.BlockSpec((1, 1), lambda b, j: (0, 0)),
            pl.BlockSpec((1, 1), lambda b, j: (0, 0)),
        ],
        out_shape=[
            jax.ShapeDtypeStruct((1, 1), jnp.float32),
            jax.ShapeDtypeStruct((1, 1), jnp.float32),
        ],
        compiler_params=pltpu.CompilerParams(
            dimension_semantics=("arbitrary", "arbitrary"),
        ),
    )(s, t, m, x)

    return sums[0, 0] / cnts[0, 0]


# speedup vs baseline: 1.0179x; 1.0179x over previous
import jax
import jax.numpy as jnp
from jax.experimental import pallas as pl
from jax.experimental.pallas import tpu as pltpu

B, D, H, W = 4, 96, 384, 384
P = H * W
PR = P // 128
RB = 144
NB = PR // RB


def _body(s_ref, t_ref, m_ref, x_ref, sum_ref, cnt_ref):
    step = pl.program_id(0) * pl.num_programs(1) + pl.program_id(1)

    @pl.when(step == 0)
    def _init():
        sum_ref[...] = jnp.zeros_like(sum_ref)
        cnt_ref[...] = jnp.zeros_like(cnt_ref)

    s = s_ref[0]
    t = t_ref[0]
    dot = jnp.sum(s * t, axis=0)
    ns2 = jnp.sum(s * s, axis=0)
    nt2 = jnp.sum(t * t, axis=0)
    denom = jnp.maximum(jnp.sqrt(ns2) * jnp.sqrt(nt2), 1e-8)
    loss_map = 1.0 - dot / denom

    valid = (x_ref[0] != 0.0) & (m_ref[0] == 0)
    vf = valid.astype(jnp.float32)
    sum_ref[...] += jnp.sum(loss_map * vf, keepdims=True).reshape(1, 1)
    cnt_ref[...] += jnp.sum(vf, keepdims=True).reshape(1, 1)


def kernel(student_feats, teacher_feats, mask, original_x):
    s = student_feats.reshape(B, D, PR, 128)
    t = teacher_feats.reshape(B, D, PR, 128)
    m = mask.reshape(B, PR, 128).astype(jnp.float32)
    x = original_x.reshape(B, PR, 128)

    sums, cnts = pl.pallas_call(
        _body,
        grid=(B, NB),
        in_specs=[
            pl.BlockSpec((1, D, RB, 128), lambda b, j: (b, 0, j, 0)),
            pl.BlockSpec((1, D, RB, 128), lambda b, j: (b, 0, j, 0)),
            pl.BlockSpec((1, RB, 128), lambda b, j: (b, j, 0)),
            pl.BlockSpec((1, RB, 128), lambda b, j: (b, j, 0)),
        ],
        out_specs=[
            pl.BlockSpec((1, 1), lambda b, j: (0, 0)),
            pl.BlockSpec((1, 1), lambda b, j: (0, 0)),
        ],
        out_shape=[
            jax.ShapeDtypeStruct((1, 1), jnp.float32),
            jax.ShapeDtypeStruct((1, 1), jnp.float32),
        ],
        compiler_params=pltpu.CompilerParams(
            dimension_semantics=("arbitrary", "arbitrary"),
        ),
    )(s, t, m, x)

    return sums[0, 0] / cnts[0, 0]
